# baseline (speedup 1.0000x reference)
import functools

import jax
import jax.numpy as jnp
from jax import lax
from jax.experimental import pallas as pl
from jax.experimental.pallas import tpu as pltpu
from jax.experimental.pallas import tpu_sc as plsc

N = 10000
E = 320000
D = 128
H = 256
DE = 16
L = 5
G = 256
FEAT = 256

NC = 2
NS = 16
NW = NC * NS
CH = 40
NCHUNK = E // CH
CHUNK_PER_W = NCHUNK // NW
NBUF = 4
NI = 8
IP = 4
ZROW = 40
NZCH = N // ZROW
ZPT = (NZCH + NS - 1) // NS


def _sc_message_pass(h, e3, eidx3):
    mesh = plsc.VectorSubcoreMesh(core_axis_name="c", subcore_axis_name="s")

    @functools.partial(
        pl.kernel,
        out_type=jax.ShapeDtypeStruct((NC, N, D), jnp.float32),
        mesh=mesh,
        scratch_types=[
            pltpu.VMEM((NI, 2, CH), jnp.int32),
            pltpu.VMEM((NBUF, CH, D), jnp.float32),
            pltpu.VMEM((NBUF, CH // 2, D), jnp.int32),
            pltpu.VMEM_SHARED((N, D), jnp.float32),
            [pltpu.SemaphoreType.DMA] * NI,
            [pltpu.SemaphoreType.DMA] * NBUF,
            [pltpu.SemaphoreType.DMA] * NBUF,
            [pltpu.SemaphoreType.DMA] * NBUF,
        ],
    )
    def body(h_hbm, e_hbm, eidx_hbm, out_hbm,
             idx_v, hbuf, mbuf, agg_sh, isems, gsems, esems, ssems):
        cid = lax.axis_index("c")
        sid = lax.axis_index("s")
        wid = cid * NS + sid
        base = wid * CHUNK_PER_W

        ivs = [idx_v.at[i] for i in range(NI)]
        hbs = [hbuf.at[i] for i in range(NBUF)]
        mbs = [mbuf.at[i] for i in range(NBUF)]

        def start_idx(k, p8):
            pltpu.async_copy(eidx_hbm.at[base + k], ivs[p8], isems[p8])

        def wait_idx(k, p8):
            pltpu.make_async_copy(eidx_hbm.at[base + k], ivs[p8],
                                  isems[p8]).wait()

        def start_fetch(k, p8, p4):
            pltpu.async_copy(h_hbm.at[ivs[p8].at[0]], hbs[p4], gsems[p4])
            pltpu.async_copy(e_hbm.at[base + k], mbs[p4], esems[p4])

        def wait_fetch(k, p8, p4):
            pltpu.make_async_copy(h_hbm.at[ivs[p8].at[0]], hbs[p4],
                                  gsems[p4]).wait()
            pltpu.make_async_copy(e_hbm.at[base + k], mbs[p4],
                                  esems[p4]).wait()

        def compute(p4):
            hb, mb = hbs[p4], mbs[p4]
            @plsc.parallel_loop(0, CH // 2)
            def _(r2):
                for cw in range(D // 16):
                    w = mb[r2, pl.ds(cw * 16, 16)]
                    lo = lax.bitcast_convert_type(
                        lax.shift_left(w, jnp.full((16,), 16, jnp.int32)),
                        jnp.float32)
                    hi = lax.bitcast_convert_type(
                        lax.bitwise_and(w, jnp.full((16,), -65536, jnp.int32)),
                        jnp.float32)
                    er = 2 * r2 + (1 if cw >= 4 else 0)
                    g = cw % 4
                    sl0 = pl.ds(g * 32, 16)
                    sl1 = pl.ds(g * 32 + 16, 16)
                    hb[er, sl0] = jnp.maximum(hb[er, sl0] + lo, 0.0)
                    hb[er, sl1] = jnp.maximum(hb[er, sl1] + hi, 0.0)

        def start_scatter(p8, p4):
            pltpu.async_copy(hbs[p4], agg_sh.at[ivs[p8].at[1]], ssems[p4],
                             add=True)

        def drain_scatter(p8, p4):
            pltpu.make_async_copy(hbs[p4], agg_sh.at[ivs[p8].at[1]],
                                  ssems[p4]).wait()

        for j in range(IP):
            start_idx(jnp.int32(j), j)

        @plsc.parallel_loop(0, ZROW)
        def _(r):
            for c8 in range(D // 16):
                hbuf[0, r, pl.ds(c8 * 16, 16)] = jnp.zeros((16,), jnp.float32)
        for k in range(ZPT):
            zc = sid + NS * k
            @pl.when(zc < NZCH)
            def _():
                pltpu.sync_copy(hbuf.at[0], agg_sh.at[pl.ds(zc * ZROW, ZROW)])
        plsc.subcore_barrier()

        for j in range(2):
            wait_idx(jnp.int32(j), j)
            start_fetch(jnp.int32(j), j, j)

        def half(k, p8, p4):
            d4 = (p4 + 2) % NBUF
            d8 = (p8 + 6) % NI
            f8 = (p8 + 2) % NI
            i8 = (p8 + IP) % NI
            @pl.when(k >= 2)
            def _():
                drain_scatter(d8, d4)
            @pl.when(k + IP < CHUNK_PER_W)
            def _():
                start_idx(k + IP, i8)
            @pl.when(k + 2 < CHUNK_PER_W)
            def _():
                wait_idx(k + 2, f8)
                start_fetch(k + 2, f8, d4)
            wait_fetch(k, p8, p4)
            compute(p4)
            start_scatter(p8, p4)

        def loop_body(k8, carry):
            for j in range(NI):
                half(NI * k8 + j, j, j % NBUF)
            return carry
        lax.fori_loop(0, CHUNK_PER_W // NI, loop_body, 0)
        for j in range(CHUNK_PER_W % NI):
            k = (CHUNK_PER_W // NI) * NI + j
            half(jnp.int32(k), k % NI, k % NBUF)

        for j in range(2):
            k = CHUNK_PER_W - 2 + j
            drain_scatter(k % NI, k % NBUF)
        plsc.subcore_barrier()

        for k in range(ZPT):
            zc = sid + NS * k
            if k >= 2:
                zcp = sid + NS * (k - 2)
                @pl.when(zcp < NZCH)
                def _():
                    pltpu.make_async_copy(hbuf.at[k % 2],
                                          out_hbm.at[cid, pl.ds(zcp * ZROW, ZROW)],
                                          gsems[k % 2]).wait()
            @pl.when(zc < NZCH)
            def _():
                pltpu.sync_copy(agg_sh.at[pl.ds(zc * ZROW, ZROW)],
                                hbuf.at[k % 2])
                pltpu.async_copy(hbuf.at[k % 2],
                                 out_hbm.at[cid, pl.ds(zc * ZROW, ZROW)],
                                 gsems[k % 2])
        for k in range(ZPT - 2, ZPT):
            zc = sid + NS * k
            @pl.when(zc < NZCH)
            def _():
                pltpu.make_async_copy(hbuf.at[k % 2],
                                      out_hbm.at[cid, pl.ds(zc * ZROW, ZROW)],
                                      gsems[k % 2]).wait()

    return body(h, e3, eidx3)


_EBLK = 8000


def _edge_encoder_body(ea_ref, we_ref, out_ref):
    out_ref[...] = jnp.dot(ea_ref[...], we_ref[...],
                           preferred_element_type=jnp.float32
                           ).astype(jnp.bfloat16)


def _edge_encoder(edge_attr, W_e_sw):
    grid = E // _EBLK
    return pl.pallas_call(
        _edge_encoder_body,
        grid=(grid,),
        in_specs=[
            pl.BlockSpec((_EBLK, DE), lambda i: (i, 0)),
            pl.BlockSpec((DE, D), lambda i: (0, 0)),
        ],
        out_specs=pl.BlockSpec((_EBLK, D), lambda i: (i, 0)),
        out_shape=jax.ShapeDtypeStruct((E, D), jnp.bfloat16),
    )(edge_attr, W_e_sw)


_RBLK = 2000


def _mlp_body(scale_ref, h_ref, agg_ref, w1_ref, b1_ref, w2_ref, b2_ref,
              out_ref, *, final_relu):
    u = scale_ref[0] * h_ref[...] + agg_ref[0] + agg_ref[1]
    t = jnp.dot(u, w1_ref[...], preferred_element_type=jnp.float32) + b1_ref[...]
    t = jnp.maximum(t, 0.0)
    z = jnp.dot(t, w2_ref[...], preferred_element_type=jnp.float32) + b2_ref[...]
    if final_relu:
        z = jnp.maximum(z, 0.0)
    out_ref[...] = z


def _mlp_layer(h, agg2, W1l, b1l, W2l, b2l, scale, final_relu):
    grid = N // _RBLK
    return pl.pallas_call(
        functools.partial(_mlp_body, final_relu=final_relu),
        grid=(grid,),
        in_specs=[
            pl.BlockSpec(memory_space=pltpu.SMEM),
            pl.BlockSpec((_RBLK, D), lambda i: (i, 0)),
            pl.BlockSpec((NC, _RBLK, D), lambda i: (0, i, 0)),
            pl.BlockSpec((D, H), lambda i: (0, 0)),
            pl.BlockSpec((1, H), lambda i: (0, 0)),
            pl.BlockSpec((H, D), lambda i: (0, 0)),
            pl.BlockSpec((1, D), lambda i: (0, 0)),
        ],
        out_specs=pl.BlockSpec((_RBLK, D), lambda i: (i, 0)),
        out_shape=jax.ShapeDtypeStruct((N, D), jnp.float32),
    )(scale, h, agg2, W1l, b1l, W2l, b2l)


def _mlp_last_body(scale_ref, h_ref, agg_ref, w1_ref, b1_ref, w2_ref, b2_ref,
                   batch_ref, out_ref, sums_ref, counts_ref):
    i = pl.program_id(0)
    u = scale_ref[0] * h_ref[...] + agg_ref[0] + agg_ref[1]
    t = jnp.dot(u, w1_ref[...], preferred_element_type=jnp.float32) + b1_ref[...]
    t = jnp.maximum(t, 0.0)
    z = jnp.dot(t, w2_ref[...], preferred_element_type=jnp.float32) + b2_ref[...]
    out_ref[...] = z

    b_blk = batch_ref[0, 0, :]
    iota_g = lax.broadcasted_iota(jnp.int32, (_RBLK, G), 1)
    onehot = (b_blk[:, None] == iota_g).astype(jnp.float32)
    part_sums = lax.dot_general(onehot, z, (((0,), (0,)), ((), ())),
                                preferred_element_type=jnp.float32)
    part_counts = jnp.sum(onehot, axis=0)[None, :]

    @pl.when(i == 0)
    def _():
        sums_ref[...] = jnp.zeros_like(sums_ref)
        counts_ref[...] = jnp.zeros_like(counts_ref)

    sums_ref[...] += part_sums
    counts_ref[...] += part_counts


def _mlp_last_layer(h, agg2, W1l, b1l, W2l, b2l, scale, batch2d):
    grid = N // _RBLK
    return pl.pallas_call(
        _mlp_last_body,
        grid=(grid,),
        in_specs=[
            pl.BlockSpec(memory_space=pltpu.SMEM),
            pl.BlockSpec((_RBLK, D), lambda i: (i, 0)),
            pl.BlockSpec((NC, _RBLK, D), lambda i: (0, i, 0)),
            pl.BlockSpec((D, H), lambda i: (0, 0)),
            pl.BlockSpec((1, H), lambda i: (0, 0)),
            pl.BlockSpec((H, D), lambda i: (0, 0)),
            pl.BlockSpec((1, D), lambda i: (0, 0)),
            pl.BlockSpec((1, 1, _RBLK), lambda i: (i, 0, 0)),
        ],
        out_specs=[
            pl.BlockSpec((_RBLK, D), lambda i: (i, 0)),
            pl.BlockSpec((G, D), lambda i: (0, 0)),
            pl.BlockSpec((1, G), lambda i: (0, 0)),
        ],
        out_shape=[
            jax.ShapeDtypeStruct((N, D), jnp.float32),
            jax.ShapeDtypeStruct((G, D), jnp.float32),
            jax.ShapeDtypeStruct((1, G), jnp.float32),
        ],
    )(scale, h, agg2, W1l, b1l, W2l, b2l, batch2d)


def _proj_body(sums_ref, counts_ref, wo_ref, bo_ref, out_ref):
    c = jnp.maximum(counts_ref[...], 1.0)
    mean = sums_ref[...] * (1.0 / c)[0, :, None]
    out_ref[...] = jnp.dot(mean, wo_ref[...],
                           preferred_element_type=jnp.float32) + bo_ref[...]


def _projection(sums, counts, W_out, b_out):
    return pl.pallas_call(
        _proj_body,
        in_specs=[
            pl.BlockSpec((G, D), lambda: (0, 0)),
            pl.BlockSpec((1, G), lambda: (0, 0)),
            pl.BlockSpec((D, FEAT), lambda: (0, 0)),
            pl.BlockSpec((1, FEAT), lambda: (0, 0)),
        ],
        out_specs=pl.BlockSpec((G, FEAT), lambda: (0, 0)),
        out_shape=jax.ShapeDtypeStruct((G, FEAT), jnp.float32),
    )(sums, counts, W_out, b_out)


def kernel(x, edge_index, edge_attr, batch, W_e, W1, b1, W2, b2, eps,
           W_out, b_out):
    eidx3 = (edge_index.astype(jnp.int32)
             .reshape(2, NCHUNK, CH).transpose(1, 0, 2))
    batch2d = batch.astype(jnp.int32).reshape(N // _RBLK, 1, _RBLK)

    perm = jnp.asarray([32 * g + 16 * (m % 2) + m // 2
                        for g in range(D // 32) for m in range(32)],
                       dtype=jnp.int32)
    W_e_sw = W_e[:, perm]

    e_bf = _edge_encoder(edge_attr, W_e_sw)
    e = (lax.bitcast_convert_type(e_bf.reshape(E, D // 2, 2), jnp.int32)
         .reshape(NCHUNK, CH // 2, D))

    h = x
    for l in range(L):
        agg2 = _sc_message_pass(h, e, eidx3)
        scale = (1.0 + eps[l]).reshape(1).astype(jnp.float32)
        if l < L - 1:
            h = _mlp_layer(h, agg2, W1[l], b1[l].reshape(1, H), W2[l],
                           b2[l].reshape(1, D), scale, final_relu=True)
        else:
            h, sums, counts = _mlp_last_layer(
                h, agg2, W1[l], b1[l].reshape(1, H), W2[l],
                b2[l].reshape(1, D), scale, batch2d)

    graph_embeds = _projection(sums, counts, W_out, b_out.reshape(1, FEAT))
    graph_mask = (counts[0] > 0.0)
    return graph_embeds, graph_mask, h


# speedup vs baseline: 1.8259x; 1.8259x over previous
import functools

import jax
import jax.numpy as jnp
from jax import lax
from jax.experimental import pallas as pl
from jax.experimental.pallas import tpu as pltpu
from jax.experimental.pallas import tpu_sc as plsc

N = 10000
E = 320000
D = 128
H = 256
DE = 16
L = 5
G = 256
FEAT = 256

NC = 2
NS = 16
NW = NC * NS
CH = 40
NCHUNK = E // CH
CHUNK_PER_W = NCHUNK // NW
NBUF = 4
NI = 8
IP = 4
ZROW = 40
NZCH = N // ZROW
ZPT = (NZCH + NS - 1) // NS


def _sc_message_pass(h, e3, eidx3):
    mesh = plsc.VectorSubcoreMesh(core_axis_name="c", subcore_axis_name="s")

    @functools.partial(
        pl.kernel,
        out_type=jax.ShapeDtypeStruct((NC, N, D), jnp.float32),
        mesh=mesh,
        scratch_types=[
            pltpu.VMEM((NI, 2, CH), jnp.int32),
            pltpu.VMEM((NBUF, CH, D), jnp.float32),
            pltpu.VMEM((NBUF, CH // 2, D), jnp.int32),
            pltpu.VMEM_SHARED((N, D), jnp.float32),
            [pltpu.SemaphoreType.DMA] * NI,
            [pltpu.SemaphoreType.DMA] * NBUF,
            [pltpu.SemaphoreType.DMA] * NBUF,
            [pltpu.SemaphoreType.DMA] * NBUF,
        ],
    )
    def body(h_hbm, e_hbm, eidx_hbm, out_hbm,
             idx_v, hbuf, mbuf, agg_sh, isems, gsems, esems, ssems):
        cid = lax.axis_index("c")
        sid = lax.axis_index("s")
        wid = cid * NS + sid
        base = wid * CHUNK_PER_W

        ivs = [idx_v.at[i] for i in range(NI)]
        hbs = [hbuf.at[i] for i in range(NBUF)]
        mbs = [mbuf.at[i] for i in range(NBUF)]

        def start_idx(k, p8):
            pltpu.async_copy(eidx_hbm.at[base + k], ivs[p8], isems[p8])

        def wait_idx(k, p8):
            pltpu.make_async_copy(eidx_hbm.at[base + k], ivs[p8],
                                  isems[p8]).wait()

        def start_fetch(k, p8, p4):
            pltpu.async_copy(h_hbm.at[ivs[p8].at[0]], hbs[p4], gsems[p4])
            pltpu.async_copy(e_hbm.at[base + k], mbs[p4], esems[p4])

        def wait_fetch(k, p8, p4):
            pltpu.make_async_copy(h_hbm.at[ivs[p8].at[0]], hbs[p4],
                                  gsems[p4]).wait()
            pltpu.make_async_copy(e_hbm.at[base + k], mbs[p4],
                                  esems[p4]).wait()

        def compute(p4):
            hb, mb = hbs[p4], mbs[p4]
            @plsc.parallel_loop(0, CH // 2)
            def _(r2):
                for cw in range(D // 16):
                    w = mb[r2, pl.ds(cw * 16, 16)]
                    lo = lax.bitcast_convert_type(
                        lax.shift_left(w, jnp.full((16,), 16, jnp.int32)),
                        jnp.float32)
                    hi = lax.bitcast_convert_type(
                        lax.bitwise_and(w, jnp.full((16,), -65536, jnp.int32)),
                        jnp.float32)
                    er = 2 * r2 + (1 if cw >= 4 else 0)
                    g = cw % 4
                    sl0 = pl.ds(g * 32, 16)
                    sl1 = pl.ds(g * 32 + 16, 16)
                    hb[er, sl0] = jnp.maximum(hb[er, sl0] + lo, 0.0)
                    hb[er, sl1] = jnp.maximum(hb[er, sl1] + hi, 0.0)

        def start_scatter(p8, p4):
            pltpu.async_copy(hbs[p4], agg_sh.at[ivs[p8].at[1]], ssems[p4],
                             add=True)

        def drain_scatter(p8, p4):
            pltpu.make_async_copy(hbs[p4], agg_sh.at[ivs[p8].at[1]],
                                  ssems[p4]).wait()

        for j in range(IP):
            start_idx(jnp.int32(j), j)

        @plsc.parallel_loop(0, ZROW)
        def _(r):
            for c8 in range(D // 16):
                hbuf[0, r, pl.ds(c8 * 16, 16)] = jnp.zeros((16,), jnp.float32)
        for k in range(ZPT):
            zc = sid + NS * k
            @pl.when(zc < NZCH)
            def _():
                pltpu.sync_copy(hbuf.at[0], agg_sh.at[pl.ds(zc * ZROW, ZROW)])
        plsc.subcore_barrier()

        for j in range(2):
            wait_idx(jnp.int32(j), j)
            start_fetch(jnp.int32(j), j, j)

        def half(k, p8, p4):
            d4 = (p4 + 2) % NBUF
            d8 = (p8 + 6) % NI
            f8 = (p8 + 2) % NI
            i8 = (p8 + IP) % NI
            @pl.when(k >= 2)
            def _():
                drain_scatter(d8, d4)
            @pl.when(k + IP < CHUNK_PER_W)
            def _():
                start_idx(k + IP, i8)
            @pl.when(k + 2 < CHUNK_PER_W)
            def _():
                wait_idx(k + 2, f8)
                start_fetch(k + 2, f8, d4)
            wait_fetch(k, p8, p4)
            compute(p4)
            start_scatter(p8, p4)

        def loop_body(k8, carry):
            for j in range(NI):
                half(NI * k8 + j, j, j % NBUF)
            return carry
        lax.fori_loop(0, CHUNK_PER_W // NI, loop_body, 0)
        for j in range(CHUNK_PER_W % NI):
            k = (CHUNK_PER_W // NI) * NI + j
            half(jnp.int32(k), k % NI, k % NBUF)

        for j in range(2):
            k = CHUNK_PER_W - 2 + j
            drain_scatter(k % NI, k % NBUF)
        plsc.subcore_barrier()

        for k in range(ZPT):
            zc = sid + NS * k
            if k >= 2:
                zcp = sid + NS * (k - 2)
                @pl.when(zcp < NZCH)
                def _():
                    pltpu.make_async_copy(hbuf.at[k % 2],
                                          out_hbm.at[cid, pl.ds(zcp * ZROW, ZROW)],
                                          gsems[k % 2]).wait()
            @pl.when(zc < NZCH)
            def _():
                pltpu.sync_copy(agg_sh.at[pl.ds(zc * ZROW, ZROW)],
                                hbuf.at[k % 2])
                pltpu.async_copy(hbuf.at[k % 2],
                                 out_hbm.at[cid, pl.ds(zc * ZROW, ZROW)],
                                 gsems[k % 2])
        for k in range(ZPT - 2, ZPT):
            zc = sid + NS * k
            @pl.when(zc < NZCH)
            def _():
                pltpu.make_async_copy(hbuf.at[k % 2],
                                      out_hbm.at[cid, pl.ds(zc * ZROW, ZROW)],
                                      gsems[k % 2]).wait()

    return body(h, e3, eidx3)


_EBLK = 8000


def _edge_encoder_body(ea_ref, wa_ref, wb_ref, out_ref):
    a = jnp.dot(ea_ref[...], wa_ref[...], preferred_element_type=jnp.float32)
    b = jnp.dot(ea_ref[...], wb_ref[...], preferred_element_type=jnp.float32)
    abits = lax.bitcast_convert_type(a, jnp.int32)
    bbits = lax.bitcast_convert_type(b, jnp.int32)
    lo = lax.shift_right_logical(abits + 0x8000, 16)
    hi = (bbits + 0x8000) & jnp.int32(-65536)
    w = (lo | hi).reshape(_EBLK // CH, CH // 2, 2, D)
    sel = lax.broadcasted_iota(jnp.int32, (_EBLK // CH, CH // 2, D), 2) < 64
    out_ref[...] = jnp.where(sel, w[:, :, 0, :], w[:, :, 1, :])


def _edge_encoder(edge_attr, W_a, W_b):
    grid = E // _EBLK
    nblk = _EBLK // CH
    return pl.pallas_call(
        _edge_encoder_body,
        grid=(grid,),
        in_specs=[
            pl.BlockSpec((_EBLK, DE), lambda i: (i, 0)),
            pl.BlockSpec((DE, D), lambda i: (0, 0)),
            pl.BlockSpec((DE, D), lambda i: (0, 0)),
        ],
        out_specs=pl.BlockSpec((nblk, CH // 2, D), lambda i: (i, 0, 0)),
        out_shape=jax.ShapeDtypeStruct((NCHUNK, CH // 2, D), jnp.int32),
    )(edge_attr, W_a, W_b)


_RBLK = 2000


def _mlp_body(scale_ref, h_ref, agg_ref, w1_ref, b1_ref, w2_ref, b2_ref,
              out_ref, *, final_relu):
    u = scale_ref[0] * h_ref[...] + agg_ref[0] + agg_ref[1]
    t = jnp.dot(u, w1_ref[...], preferred_element_type=jnp.float32) + b1_ref[...]
    t = jnp.maximum(t, 0.0)
    z = jnp.dot(t, w2_ref[...], preferred_element_type=jnp.float32) + b2_ref[...]
    if final_relu:
        z = jnp.maximum(z, 0.0)
    out_ref[...] = z


def _mlp_layer(h, agg2, W1l, b1l, W2l, b2l, scale, final_relu):
    grid = N // _RBLK
    return pl.pallas_call(
        functools.partial(_mlp_body, final_relu=final_relu),
        grid=(grid,),
        in_specs=[
            pl.BlockSpec(memory_space=pltpu.SMEM),
            pl.BlockSpec((_RBLK, D), lambda i: (i, 0)),
            pl.BlockSpec((NC, _RBLK, D), lambda i: (0, i, 0)),
            pl.BlockSpec((D, H), lambda i: (0, 0)),
            pl.BlockSpec((1, H), lambda i: (0, 0)),
            pl.BlockSpec((H, D), lambda i: (0, 0)),
            pl.BlockSpec((1, D), lambda i: (0, 0)),
        ],
        out_specs=pl.BlockSpec((_RBLK, D), lambda i: (i, 0)),
        out_shape=jax.ShapeDtypeStruct((N, D), jnp.float32),
    )(scale, h, agg2, W1l, b1l, W2l, b2l)


def _mlp_last_body(scale_ref, h_ref, agg_ref, w1_ref, b1_ref, w2_ref, b2_ref,
                   batch_ref, out_ref, sums_ref, counts_ref):
    i = pl.program_id(0)
    u = scale_ref[0] * h_ref[...] + agg_ref[0] + agg_ref[1]
    t = jnp.dot(u, w1_ref[...], preferred_element_type=jnp.float32) + b1_ref[...]
    t = jnp.maximum(t, 0.0)
    z = jnp.dot(t, w2_ref[...], preferred_element_type=jnp.float32) + b2_ref[...]
    out_ref[...] = z

    b_blk = batch_ref[0, 0, :]
    iota_g = lax.broadcasted_iota(jnp.int32, (_RBLK, G), 1)
    onehot = (b_blk[:, None] == iota_g).astype(jnp.float32)
    part_sums = lax.dot_general(onehot, z, (((0,), (0,)), ((), ())),
                                preferred_element_type=jnp.float32)
    part_counts = jnp.sum(onehot, axis=0)[None, :]

    @pl.when(i == 0)
    def _():
        sums_ref[...] = jnp.zeros_like(sums_ref)
        counts_ref[...] = jnp.zeros_like(counts_ref)

    sums_ref[...] += part_sums
    counts_ref[...] += part_counts


def _mlp_last_layer(h, agg2, W1l, b1l, W2l, b2l, scale, batch2d):
    grid = N // _RBLK
    return pl.pallas_call(
        _mlp_last_body,
        grid=(grid,),
        in_specs=[
            pl.BlockSpec(memory_space=pltpu.SMEM),
            pl.BlockSpec((_RBLK, D), lambda i: (i, 0)),
            pl.BlockSpec((NC, _RBLK, D), lambda i: (0, i, 0)),
            pl.BlockSpec((D, H), lambda i: (0, 0)),
            pl.BlockSpec((1, H), lambda i: (0, 0)),
            pl.BlockSpec((H, D), lambda i: (0, 0)),
            pl.BlockSpec((1, D), lambda i: (0, 0)),
            pl.BlockSpec((1, 1, _RBLK), lambda i: (i, 0, 0)),
        ],
        out_specs=[
            pl.BlockSpec((_RBLK, D), lambda i: (i, 0)),
            pl.BlockSpec((G, D), lambda i: (0, 0)),
            pl.BlockSpec((1, G), lambda i: (0, 0)),
        ],
        out_shape=[
            jax.ShapeDtypeStruct((N, D), jnp.float32),
            jax.ShapeDtypeStruct((G, D), jnp.float32),
            jax.ShapeDtypeStruct((1, G), jnp.float32),
        ],
    )(scale, h, agg2, W1l, b1l, W2l, b2l, batch2d)


def _proj_body(sums_ref, counts_ref, wo_ref, bo_ref, out_ref):
    c = jnp.maximum(counts_ref[...], 1.0)
    mean = sums_ref[...] * (1.0 / c)[0, :, None]
    out_ref[...] = jnp.dot(mean, wo_ref[...],
                           preferred_element_type=jnp.float32) + bo_ref[...]


def _projection(sums, counts, W_out, b_out):
    return pl.pallas_call(
        _proj_body,
        in_specs=[
            pl.BlockSpec((G, D), lambda: (0, 0)),
            pl.BlockSpec((1, G), lambda: (0, 0)),
            pl.BlockSpec((D, FEAT), lambda: (0, 0)),
            pl.BlockSpec((1, FEAT), lambda: (0, 0)),
        ],
        out_specs=pl.BlockSpec((G, FEAT), lambda: (0, 0)),
        out_shape=jax.ShapeDtypeStruct((G, FEAT), jnp.float32),
    )(sums, counts, W_out, b_out)


def kernel(x, edge_index, edge_attr, batch, W_e, W1, b1, W2, b2, eps,
           W_out, b_out):
    eidx3 = (edge_index.astype(jnp.int32)
             .reshape(2, NCHUNK, CH).transpose(1, 0, 2))
    batch2d = batch.astype(jnp.int32).reshape(N // _RBLK, 1, _RBLK)

    aperm = jnp.asarray([32 * ((l % 64) // 16) + (l % 16) for l in range(D)],
                        dtype=jnp.int32)
    W_a = W_e[:, aperm]
    W_b = W_e[:, aperm + 16]

    e = _edge_encoder(edge_attr, W_a, W_b)

    h = x
    for l in range(L):
        agg2 = _sc_message_pass(h, e, eidx3)
        scale = (1.0 + eps[l]).reshape(1).astype(jnp.float32)
        if l < L - 1:
            h = _mlp_layer(h, agg2, W1[l], b1[l].reshape(1, H), W2[l],
                           b2[l].reshape(1, D), scale, final_relu=True)
        else:
            h, sums, counts = _mlp_last_layer(
                h, agg2, W1[l], b1[l].reshape(1, H), W2[l],
                b2[l].reshape(1, D), scale, batch2d)

    graph_embeds = _projection(sums, counts, W_out, b_out.reshape(1, FEAT))
    graph_mask = (counts[0] > 0.0)
    return graph_embeds, graph_mask, h
